# baseline (speedup 1.0000x reference)
import functools

import jax
import jax.numpy as jnp
from jax import lax
from jax.experimental import pallas as pl
from jax.experimental.pallas import tpu as pltpu
from jax.experimental.pallas import tpu_sc as plsc

Qn = 1024
DIN = 256
DM = 64
Kn = 100000
BK = 4096
NB = (Kn + BK - 1) // BK
CMASK = BK - 1
BLUR_SQ = 0.81


def _tc_body(x_ref, w_ref, b_ref, cols_ref, keys_ref, idx_ref, sqv_ref,
             faug_ref, qsq_ref, bm_ref, bj_ref):
    j = pl.program_id(0)

    @pl.when(j == 0)
    def _init():
        f = jax.nn.gelu(
            jnp.dot(x_ref[...], w_ref[...], preferred_element_type=jnp.float32)
            + b_ref[...])
        faug_ref[:, :DM] = f * (-2.0)
        faug_ref[:, DM:] = jnp.ones((Qn, 1), jnp.float32)
        qsq_ref[...] = jnp.sum(f * f, axis=1, keepdims=True)

    kb = keys_ref[...]
    ksq_row = jnp.sum(kb * kb, axis=0, keepdims=True)
    k_aug = jnp.concatenate([kb, ksq_row], axis=0)
    m = lax.dot_general(faug_ref[...], k_aug, (((1,), (0,)), ((), ())),
                        preferred_element_type=jnp.float32)

    z = lax.bitcast_convert_type(
        (lax.bitcast_convert_type(m, jnp.int32) & ~CMASK) | cols_ref[...],
        jnp.float32)
    zmin = jnp.min(z, axis=1, keepdims=True)
    bm_old = jnp.where(j == 0, jnp.float32(jnp.inf), bm_ref[...])
    bj_old = jnp.where(j == 0, jnp.float32(0.0), bj_ref[...])
    upd = zmin < bm_old
    bm_ref[...] = jnp.where(upd, zmin, bm_old)
    bj_ref[...] = jnp.where(upd, jnp.float32(j), bj_old)

    @pl.when(j == NB - 1)
    def _fin():
        zi = lax.bitcast_convert_type(bm_ref[...], jnp.int32)
        col = (zi & CMASK).astype(jnp.float32)
        idx_ref[...] = (bj_ref[...] * jnp.float32(BK) + col).astype(jnp.int32)
        sqv_ref[...] = (qsq_ref[...]
                        + lax.bitcast_convert_type(zi & ~CMASK, jnp.float32))


def _tc_search(x, W, b2, cols, keysT):
    return pl.pallas_call(
        _tc_body,
        grid=(NB,),
        in_specs=[
            pl.BlockSpec((Qn, DIN), lambda j: (0, 0)),
            pl.BlockSpec((DIN, DM), lambda j: (0, 0)),
            pl.BlockSpec((1, DM), lambda j: (0, 0)),
            pl.BlockSpec((1, BK), lambda j: (0, 0)),
            pl.BlockSpec((DM, BK), lambda j: (0, j)),
        ],
        out_specs=[
            pl.BlockSpec((Qn, 1), lambda j: (0, 0)),
            pl.BlockSpec((Qn, 1), lambda j: (0, 0)),
        ],
        out_shape=[
            jax.ShapeDtypeStruct((Qn, 1), jnp.int32),
            jax.ShapeDtypeStruct((Qn, 1), jnp.float32),
        ],
        scratch_shapes=[
            pltpu.VMEM((Qn, DM + 1), jnp.float32),
            pltpu.VMEM((Qn, 1), jnp.float32),
            pltpu.VMEM((Qn, 1), jnp.float32),
            pltpu.VMEM((Qn, 1), jnp.float32),
        ],
        compiler_params=pltpu.CompilerParams(
            dimension_semantics=("arbitrary",)),
    )(x, W, b2, cols, keysT)


def _sc_finish(values, idx, sqv):
    info = plsc.get_sparse_core_info()
    nw = info.num_cores * info.num_subcores
    bpw = Qn // nw
    mesh = plsc.VectorSubcoreMesh(core_axis_name="c", subcore_axis_name="s")

    @functools.partial(
        pl.kernel, mesh=mesh,
        out_type=jax.ShapeDtypeStruct((Qn,), jnp.float32),
        scratch_types=[
            pltpu.VMEM((bpw,), jnp.int32),
            pltpu.VMEM((bpw,), jnp.float32),
            pltpu.VMEM((bpw,), jnp.float32),
            pltpu.VMEM((bpw,), jnp.float32),
            pltpu.SemaphoreType.DMA,
            pltpu.SemaphoreType.DMA,
        ],
    )
    def k(values_hbm, idx_hbm, sqv_hbm, out_hbm,
          idx_v, sqv_v, vals_v, out_v, sem, sem2):
        wid = lax.axis_index("s") * info.num_cores + lax.axis_index("c")
        base = wid * bpw
        c1 = pltpu.async_copy(idx_hbm.at[pl.ds(base, bpw)], idx_v, sem)
        c2 = pltpu.async_copy(sqv_hbm.at[pl.ds(base, bpw)], sqv_v, sem2)
        c1.wait()
        pltpu.async_copy(values_hbm.at[idx_v], vals_v, sem).wait()
        c2.wait()
        for t in range(bpw // 16):
            sl = pl.ds(t * 16, 16)
            out_v[sl] = jnp.where(sqv_v[sl] <= BLUR_SQ, vals_v[sl],
                                  jnp.zeros((16,), jnp.float32))
        pltpu.sync_copy(out_v, out_hbm.at[pl.ds(base, bpw)])

    return k(values, idx, sqv)


def kernel(x, keys, values, W, b):
    cols = lax.broadcasted_iota(jnp.int32, (1, BK), 1)
    keys_t = jnp.pad(keys.T, ((0, 0), (0, NB * BK - Kn)),
                     constant_values=1e9)
    idx, sqv = _tc_search(x, W, b.reshape(1, DM), cols, keys_t)
    return _sc_finish(values, idx[:, 0], sqv[:, 0])


# speedup vs baseline: 1.1887x; 1.1887x over previous
import functools

import jax
import jax.numpy as jnp
from jax import lax
from jax.experimental import pallas as pl
from jax.experimental.pallas import tpu as pltpu
from jax.experimental.pallas import tpu_sc as plsc

Qn = 1024
DIN = 256
DM = 64
Kn = 100000
BK = 4096
NB = (Kn + BK - 1) // BK
CMASK = BK - 1
BLUR_SQ = 0.81


def _tc_body(x_ref, w_ref, b_ref, cols_ref, keys_ref, idx_ref, sqv_ref,
             faug_ref, qsq_ref, bm_ref, bj_ref):
    j = pl.program_id(0)

    @pl.when(j == 0)
    def _init():
        f = jax.nn.gelu(
            jnp.dot(x_ref[...], w_ref[...], preferred_element_type=jnp.float32)
            + b_ref[...])
        faug_ref[:, :DM] = f * (-2.0)
        faug_ref[:, DM:] = jnp.ones((Qn, 1), jnp.float32)
        qsq_ref[...] = jnp.sum(f * f, axis=1, keepdims=True)

    kb = keys_ref[...]
    lim = Kn - j * BK
    valid2 = lax.broadcasted_iota(jnp.int32, (DM, BK), 1) < lim
    kb = jnp.where(valid2, kb, 0.0)
    valid1 = lax.broadcasted_iota(jnp.int32, (1, BK), 1) < lim
    ksq_row = jnp.where(valid1,
                        jnp.sum(kb * kb, axis=0, keepdims=True),
                        jnp.float32(1e9))
    k_aug = jnp.concatenate([kb, ksq_row], axis=0)
    m = lax.dot_general(faug_ref[...], k_aug, (((1,), (0,)), ((), ())),
                        preferred_element_type=jnp.float32)

    z = lax.bitcast_convert_type(
        (lax.bitcast_convert_type(m, jnp.int32) & ~CMASK) | cols_ref[...],
        jnp.float32)
    zmin = jnp.min(z, axis=1, keepdims=True)
    bm_old = jnp.where(j == 0, jnp.float32(jnp.inf), bm_ref[...])
    bj_old = jnp.where(j == 0, jnp.float32(0.0), bj_ref[...])
    upd = zmin < bm_old
    bm_ref[...] = jnp.where(upd, zmin, bm_old)
    bj_ref[...] = jnp.where(upd, jnp.float32(j), bj_old)

    @pl.when(j == NB - 1)
    def _fin():
        zi = lax.bitcast_convert_type(bm_ref[...], jnp.int32)
        col = (zi & CMASK).astype(jnp.float32)
        idx_ref[...] = (bj_ref[...] * jnp.float32(BK) + col).astype(jnp.int32)
        sqv_ref[...] = (qsq_ref[...]
                        + lax.bitcast_convert_type(zi & ~CMASK, jnp.float32))


def _tc_search(x, W, b2, cols, keysT):
    return pl.pallas_call(
        _tc_body,
        grid=(NB,),
        in_specs=[
            pl.BlockSpec((Qn, DIN), lambda j: (0, 0)),
            pl.BlockSpec((DIN, DM), lambda j: (0, 0)),
            pl.BlockSpec((1, DM), lambda j: (0, 0)),
            pl.BlockSpec((1, BK), lambda j: (0, 0)),
            pl.BlockSpec((DM, BK), lambda j: (0, j)),
        ],
        out_specs=[
            pl.BlockSpec((Qn, 1), lambda j: (0, 0)),
            pl.BlockSpec((Qn, 1), lambda j: (0, 0)),
        ],
        out_shape=[
            jax.ShapeDtypeStruct((Qn, 1), jnp.int32),
            jax.ShapeDtypeStruct((Qn, 1), jnp.float32),
        ],
        scratch_shapes=[
            pltpu.VMEM((Qn, DM + 1), jnp.float32),
            pltpu.VMEM((Qn, 1), jnp.float32),
            pltpu.VMEM((Qn, 1), jnp.float32),
            pltpu.VMEM((Qn, 1), jnp.float32),
        ],
        compiler_params=pltpu.CompilerParams(
            dimension_semantics=("arbitrary",)),
    )(x, W, b2, cols, keysT)


def _sc_finish(values, idx, sqv):
    info = plsc.get_sparse_core_info()
    nw = info.num_cores * info.num_subcores
    bpw = Qn // nw
    mesh = plsc.VectorSubcoreMesh(core_axis_name="c", subcore_axis_name="s")

    @functools.partial(
        pl.kernel, mesh=mesh,
        out_type=jax.ShapeDtypeStruct((Qn,), jnp.float32),
        scratch_types=[
            pltpu.VMEM((bpw,), jnp.int32),
            pltpu.VMEM((bpw,), jnp.float32),
            pltpu.VMEM((bpw,), jnp.float32),
            pltpu.VMEM((bpw,), jnp.float32),
            pltpu.SemaphoreType.DMA,
            pltpu.SemaphoreType.DMA,
        ],
    )
    def k(values_hbm, idx_hbm, sqv_hbm, out_hbm,
          idx_v, sqv_v, vals_v, out_v, sem, sem2):
        wid = lax.axis_index("s") * info.num_cores + lax.axis_index("c")
        base = wid * bpw
        c1 = pltpu.async_copy(idx_hbm.at[pl.ds(base, bpw)], idx_v, sem)
        c2 = pltpu.async_copy(sqv_hbm.at[pl.ds(base, bpw)], sqv_v, sem2)
        c1.wait()
        pltpu.async_copy(values_hbm.at[idx_v], vals_v, sem).wait()
        c2.wait()
        for t in range(bpw // 16):
            sl = pl.ds(t * 16, 16)
            out_v[sl] = jnp.where(sqv_v[sl] <= BLUR_SQ, vals_v[sl],
                                  jnp.zeros((16,), jnp.float32))
        pltpu.sync_copy(out_v, out_hbm.at[pl.ds(base, bpw)])

    return k(values, idx, sqv)


def kernel(x, keys, values, W, b):
    cols = lax.broadcasted_iota(jnp.int32, (1, BK), 1)
    idx, sqv = _tc_search(x, W, b.reshape(1, DM), cols, keys.T)
    return _sc_finish(values, idx[:, 0], sqv[:, 0])
